# baseline (speedup 1.0000x reference)
import functools

import jax
import jax.numpy as jnp
from jax import lax
from jax.experimental import pallas as pl
from jax.experimental.pallas import tpu as pltpu
from jax.experimental.pallas import tpu_sc as plsc

ROWS = 128
COLS = 32768
NC = 1
NS = 16
L = 16
NW = NC * NS
K = 8
OSTEPS = COLS // (L * K)

SC_ROWS = 16
SC_BASE = ROWS - SC_ROWS
RPW = SC_ROWS // NW

TC_ROWS = ROWS - SC_ROWS
BR = 56

_mesh = plsc.VectorSubcoreMesh(core_axis_name="c", subcore_axis_name="s", num_cores=NC)

_NEG_INF = float("-inf")


@functools.partial(
    pl.kernel,
    mesh=_mesh,
    out_type=jax.ShapeDtypeStruct((NW, L), jnp.int32),
    scratch_types=[
        pltpu.VMEM((2, COLS), jnp.float32),
        pltpu.VMEM((L,), jnp.int32),
        pltpu.SemaphoreType.DMA,
        pltpu.SemaphoreType.DMA,
    ],
)
def _argmax_sc(x_hbm, out_hbm, buf, res, sem0, sem1):
    cid = lax.axis_index("c")
    sid = lax.axis_index("s")
    wid = cid * NS + sid
    base = wid * RPW
    sems = (sem0, sem1)

    copies = [pltpu.async_copy(x_hbm.at[base], buf.at[0], sems[0])]
    iota = lax.iota(jnp.int32, L)
    ansvec = jnp.zeros((L,), jnp.int32)

    for r in range(RPW):
        if r + 1 < RPW:
            copies.append(
                pltpu.async_copy(
                    x_hbm.at[base + (r + 1)], buf.at[(r + 1) % 2], sems[(r + 1) % 2]
                )
            )
        copies[r].wait()
        row = buf.at[r % 2]

        def body(jo, carry):
            maxs, steps = carry
            new_maxs = []
            new_steps = []
            for k in range(K):
                v = row[pl.ds((jo * K + k) * L, L)]
                m = v > maxs[k]
                new_maxs.append(jnp.where(m, v, maxs[k]))
                new_steps.append(jnp.where(m, jo, steps[k]))
            return tuple(new_maxs), tuple(new_steps)

        init = (
            tuple(jnp.full((L,), _NEG_INF, jnp.float32) for _ in range(K)),
            tuple(jnp.zeros((L,), jnp.int32) for _ in range(K)),
        )
        maxs, steps = lax.fori_loop(0, OSTEPS, body, init, unroll=2)

        vals = list(maxs)
        idxs = [(steps[k] * K + k) * L + iota for k in range(K)]
        n = K
        while n > 1:
            half = n // 2
            for a in range(half):
                b = a + half
                take = (vals[b] > vals[a]) | (
                    (vals[b] == vals[a]) & (idxs[b] < idxs[a])
                )
                vals[a] = jnp.where(take, vals[b], vals[a])
                idxs[a] = jnp.where(take, idxs[b], idxs[a])
            n = half
        vmax, vidx = vals[0], idxs[0]

        gmax = vmax
        for shift in (1, 2, 4, 8):
            perm = iota ^ shift
            gmax = jnp.maximum(gmax, gmax.at[perm].get(mode="promise_in_bounds"))
        cand = jnp.where(vmax == gmax, vidx, COLS)
        for shift in (1, 2, 4, 8):
            perm = iota ^ shift
            cand = jnp.minimum(cand, cand.at[perm].get(mode="promise_in_bounds"))
        ansvec = jnp.where(iota == r, cand, ansvec)

    res[...] = ansvec
    pltpu.sync_copy(res, out_hbm.at[wid])


def _tc_body(x_ref, o_ref):
    xb = x_ref[...]
    m = jnp.max(xb, axis=1, keepdims=True)
    iota = lax.broadcasted_iota(jnp.int32, (BR, COLS), 1)
    idx = jnp.where(xb == m, iota, COLS)
    o_ref[0, 0, :] = jnp.min(idx, axis=1)


def _argmax_tc(x):
    nb = TC_ROWS // BR
    out = pl.pallas_call(
        _tc_body,
        grid=(nb,),
        in_specs=[pl.BlockSpec((BR, COLS), lambda i: (i, 0))],
        out_specs=pl.BlockSpec((1, 1, BR), lambda i: (i, 0, 0)),
        out_shape=jax.ShapeDtypeStruct((nb, 1, BR), jnp.int32),
    )(x)
    return out.reshape(TC_ROWS)


def kernel(x):
    x_tail = lax.slice(x, (SC_BASE, 0), (ROWS, COLS))
    sc_out = _argmax_sc(x_tail)
    tc_out = _argmax_tc(x)
    sc_idx = sc_out[:, :RPW].reshape(SC_ROWS)
    return jnp.concatenate([tc_out, sc_idx])


# speedup vs baseline: 2.6370x; 2.6370x over previous
import jax
import jax.numpy as jnp
from jax import lax
from jax.experimental import pallas as pl

ROWS, COLS = 128, 32768
BR = 64


def _tc_body(x_ref, o_ref):
    xb = x_ref[...]
    m = jnp.max(xb, axis=1, keepdims=True)
    iota = lax.broadcasted_iota(jnp.int32, (BR, COLS), 1)
    idx = jnp.where(xb == m, iota, COLS)
    o_ref[0, 0, :] = jnp.min(idx, axis=1)


def _argmax_tc(x):
    nb = ROWS // BR
    out = pl.pallas_call(
        _tc_body,
        grid=(nb,),
        in_specs=[pl.BlockSpec((BR, COLS), lambda i: (i, 0))],
        out_specs=pl.BlockSpec((1, 1, BR), lambda i: (i, 0, 0)),
        out_shape=jax.ShapeDtypeStruct((nb, 1, BR), jnp.int32),
    )(x)
    return out.reshape(ROWS)


def kernel(x):
    return _argmax_tc(x)
